# baseline (speedup 1.0000x reference)
import jax
import jax.numpy as jnp
from jax.experimental import pallas as pl
from jax.experimental.pallas import tpu as pltpu

B = 64
K = 32768
HIDDEN = 1024
NUM_LABELS = 3
TOP_K = 8
END_K = 8
T = 0.07
RATE = 0.5

CHUNK = 2048
NCHUNK = K // CHUNK
CB = 16384
NB = K // CB
KEY_MIN = -2147483648
KEY_MAX = 2147483647
OUT_PAD = 128


def _to_key(x):
    bits = jax.lax.bitcast_convert_type(x, jnp.int32)
    return jnp.where(bits < 0, bits ^ jnp.int32(0x7FFFFFFF), bits)


def _from_key(k):
    bits = jnp.where(k < 0, k ^ jnp.int32(0x7FFFFFFF), k)
    return jax.lax.bitcast_convert_type(bits, jnp.float32)


def _heads_body(feat_ref, labels_ref, cdw_ref, cdb_ref, cow_ref, cob_ref,
                kdw_ref, kdb_ref, kow_ref, kob_ref, lq_ref, lcls_ref):
    f = feat_ref[...]
    h = jnp.tanh(
        jax.lax.dot_general(f, kdw_ref[...], (((1,), (0,)), ((), ())),
                            preferred_element_type=jnp.float32)
        + kdb_ref[...])
    logits = jax.lax.dot_general(h, kow_ref[...], (((1,), (0,)), ((), ())),
                                 preferred_element_type=jnp.float32)
    logits = logits + kob_ref[...]
    col = jax.lax.broadcasted_iota(jnp.int32, (B, OUT_PAD), 1)
    valid = col < NUM_LABELS
    logits = jnp.where(valid, logits, -3e38)
    m = jnp.max(logits, axis=1, keepdims=True)
    lse = jnp.log(jnp.sum(jnp.exp(logits - m), axis=1, keepdims=True)) + m
    onehot = col == labels_ref[...]
    pick = jnp.sum(jnp.where(onehot, logits, 0.0), axis=1, keepdims=True)
    lcls_ref[...] = jnp.reshape(jnp.mean(lse - pick), (1, 1))
    c = jnp.tanh(
        jax.lax.dot_general(f, cdw_ref[...], (((1,), (0,)), ((), ())),
                            preferred_element_type=jnp.float32)
        + cdb_ref[...])
    q = jax.lax.dot_general(c, cow_ref[...], (((1,), (0,)), ((), ())),
                            preferred_element_type=jnp.float32) + cob_ref[...]
    lq_ref[...] = q / jnp.sqrt(jnp.sum(q * q, axis=1, keepdims=True))


def _con_body(lq_ref, labels_ref, fq_ref, lblq_ref, lcon_ref, upos_ref,
              uneg_ref):
    i = pl.program_id(0)
    cos = jax.lax.dot_general(lq_ref[...], fq_ref[...],
                              (((1,), (1,)), ((), ())),
                              preferred_element_type=jnp.float32)
    key = _to_key(cos)
    mask = lblq_ref[...] == labels_ref[...]
    off = i * CHUNK
    upos_ref[:, pl.ds(off, CHUNK)] = jnp.where(mask, key, KEY_MIN)
    uneg_ref[:, pl.ds(off, CHUNK)] = jnp.where(mask, KEY_MIN, key)

    @pl.when(i == NCHUNK - 1)
    def _selection():
        z = jnp.zeros((B, 1), jnp.float32)

        def cnt_blk(j, c):
            u = upos_ref[:, pl.ds(j * CB, CB)]
            return c + jnp.sum(jnp.where(u != KEY_MIN, 1.0, 0.0), axis=1,
                               keepdims=True)
        cpos = jax.lax.fori_loop(0, NB, cnt_blk, z)
        pos_min = jnp.min(cpos)
        neg_min = jnp.min(float(K) - cpos)
        start = jnp.clip(pos_min - END_K, 0.0, float(K - END_K))
        kpos = start + END_K
        kneg = neg_min

        pp = jnp.full((B, 1), KEY_MIN, jnp.int32)
        pn = jnp.full((B, 1), KEY_MIN, jnp.int32)
        for bit in range(30, -1, -1):
            step = jnp.int32(1 << bit)
            tp = pp + step
            tn = pn + step
            def blk(j, c, tp=tp, tn=tn):
                cp, cn = c
                up = upos_ref[:, pl.ds(j * CB, CB)]
                un = uneg_ref[:, pl.ds(j * CB, CB)]
                cp = cp + jnp.sum(jnp.where(up >= tp, 1.0, 0.0), axis=1,
                                  keepdims=True)
                cn = cn + jnp.sum(jnp.where(un >= tn, 1.0, 0.0), axis=1,
                                  keepdims=True)
                return cp, cn
            cp, cn = jax.lax.fori_loop(0, NB, blk, (z, z))
            pp = jnp.where(cp >= kpos, tp, pp)
            pn = jnp.where(cn >= kneg, tn, pn)
        tpos, tneg = pp, pn

        def passa(j, c):
            gp, gn, mk, m0, n0 = c
            up = upos_ref[:, pl.ds(j * CB, CB)]
            un = uneg_ref[:, pl.ds(j * CB, CB)]
            gp = gp + jnp.sum(jnp.where(up > tpos, 1.0, 0.0), axis=1,
                              keepdims=True)
            gn = gn + jnp.sum(jnp.where(un > tneg, 1.0, 0.0), axis=1,
                              keepdims=True)
            mk = jnp.maximum(mk, jnp.max(un, axis=1, keepdims=True))
            m0 = jnp.maximum(m0, jnp.max(up, axis=1, keepdims=True))
            n0 = jnp.minimum(n0, jnp.min(
                jnp.where(up > tpos, up, KEY_MAX), axis=1, keepdims=True))
            return gp, gn, mk, m0, n0
        gpos, gneg, mkneg, m_cur, n_cur = jax.lax.fori_loop(
            0, NB, passa,
            (z, z, jnp.full((B, 1), KEY_MIN, jnp.int32),
             jnp.full((B, 1), KEY_MIN, jnp.int32),
             jnp.full((B, 1), KEY_MAX, jnp.int32)))
        M = _from_key(jnp.maximum(mkneg, m_cur))

        def passb(j, e):
            un = uneg_ref[:, pl.ds(j * CB, CB)]
            x = _from_key(un)
            t = jnp.where(un > tneg, jnp.exp((x - M) / T), 0.0)
            return e + jnp.sum(t, axis=1, keepdims=True)
        esum = jax.lax.fori_loop(0, NB, passb, z)
        tneg_val = _from_key(tneg)
        esum = esum + (kneg - gneg) * jnp.exp((tneg_val - M) / T)

        mvals, mcnts, nvals, ncnts = [], [], [], []
        for _ in range(TOP_K):
            def passc(j, c, m_cur=m_cur, n_cur=n_cur):
                cm, nm, cn_, nn = c
                up = upos_ref[:, pl.ds(j * CB, CB)]
                cm = cm + jnp.sum(jnp.where(up == m_cur, 1.0, 0.0), axis=1,
                                  keepdims=True)
                nm = jnp.maximum(nm, jnp.max(
                    jnp.where(up < m_cur, up, KEY_MIN), axis=1,
                    keepdims=True))
                cn_ = cn_ + jnp.sum(jnp.where(up == n_cur, 1.0, 0.0), axis=1,
                                    keepdims=True)
                nn = jnp.minimum(nn, jnp.min(
                    jnp.where(up > n_cur, up, KEY_MAX), axis=1,
                    keepdims=True))
                return cm, nm, cn_, nn
            cm, m_next, cn_, n_next = jax.lax.fori_loop(
                0, NB, passc,
                (z, jnp.full((B, 1), KEY_MIN, jnp.int32), z,
                 jnp.full((B, 1), KEY_MAX, jnp.int32)))
            mvals.append(m_cur); mcnts.append(cm)
            nvals.append(n_cur); ncnts.append(cn_)
            m_cur, n_cur = m_next, n_next

        mval = jnp.concatenate(mvals, axis=1)
        nval = jnp.concatenate(nvals, axis=1)
        mcum_l, ncum_l = [], []
        racc_m = jnp.zeros((B, 1), jnp.float32)
        racc_n = jnp.zeros((B, 1), jnp.float32)
        for q in range(TOP_K):
            racc_m = racc_m + mcnts[q]
            racc_n = racc_n + ncnts[q]
            mcum_l.append(racc_m)
            ncum_l.append(racc_n)
        mcum = jnp.concatenate(mcum_l, axis=1)
        ncum = jnp.concatenate(ncum_l, axis=1)
        mprev = mcum - jnp.concatenate(mcnts, axis=1)
        nprev = ncum - jnp.concatenate(ncnts, axis=1)
        mf = _from_key(mval)
        nf = _from_key(nval)
        tpos_val = _from_key(tpos)
        mt = kpos - gpos

        contrib = jnp.zeros((B, 1), jnp.float32)
        for j in range(TOP_K):
            jf = float(j)
            selm = (mprev <= jf) & (mcum > jf)
            vj = jnp.sum(jnp.where(selm, mf, 0.0), axis=1, keepdims=True)
            d = (vj - M) / T
            contrib = contrib + jnp.log(jnp.exp(d) + esum) - d
            seln = (nprev <= jf) & (ncum > jf)
            aj = jnp.sum(jnp.where(seln, nf, 0.0), axis=1, keepdims=True)
            wj = jnp.where(jf < END_K - mt, aj, tpos_val)
            d2 = (wj - M) / T
            contrib = contrib + jnp.log(jnp.exp(d2) + esum) - d2
        lcon_ref[...] = jnp.reshape(
            jnp.sum(contrib) / (B * (TOP_K + END_K)), (1, 1))


@jax.jit
def kernel(features, feature_queue, labels, label_queue, cls_dense_w,
           cls_dense_b, cls_out_w, cls_out_b, con_dense_w, con_dense_b,
           con_out_w, con_out_b):
    labels2 = labels.astype(jnp.int32).reshape(B, 1)
    lblq = label_queue.astype(jnp.int32).reshape(1, K)
    kow = jnp.zeros((HIDDEN, OUT_PAD), jnp.float32).at[:, :NUM_LABELS].set(
        cls_out_w)
    kob = jnp.zeros((1, OUT_PAD), jnp.float32).at[0, :NUM_LABELS].set(
        cls_out_b)

    lq, lcls = pl.pallas_call(
        _heads_body,
        out_shape=(jax.ShapeDtypeStruct((B, HIDDEN), jnp.float32),
                   jax.ShapeDtypeStruct((1, 1), jnp.float32)),
    )(features, labels2, con_dense_w, con_dense_b.reshape(1, HIDDEN),
      con_out_w, con_out_b.reshape(1, HIDDEN), cls_dense_w,
      cls_dense_b.reshape(1, HIDDEN), kow, kob)

    lcon = pl.pallas_call(
        _con_body,
        grid=(NCHUNK,),
        in_specs=[
            pl.BlockSpec((B, HIDDEN), lambda i: (0, 0)),
            pl.BlockSpec((B, 1), lambda i: (0, 0)),
            pl.BlockSpec((CHUNK, HIDDEN), lambda i: (i, 0)),
            pl.BlockSpec((1, CHUNK), lambda i: (0, i)),
        ],
        out_specs=pl.BlockSpec((1, 1), lambda i: (0, 0)),
        out_shape=jax.ShapeDtypeStruct((1, 1), jnp.float32),
        scratch_shapes=[pltpu.VMEM((B, K), jnp.int32),
                        pltpu.VMEM((B, K), jnp.int32)],
        compiler_params=pltpu.CompilerParams(
            dimension_semantics=("arbitrary",)),
    )(lq, labels2, feature_queue, lblq)

    return (RATE * lcon[0, 0] + (1.0 - RATE) * lcls[0, 0]).reshape(())


# speedup vs baseline: 1.2186x; 1.2186x over previous
import jax
import jax.numpy as jnp
from jax.experimental import pallas as pl
from jax.experimental.pallas import tpu as pltpu

B = 64
K = 32768
HIDDEN = 1024
NUM_LABELS = 3
TOP_K = 8
END_K = 8
T = 0.07
RATE = 0.5

CHUNK = 2048
NCHUNK = K // CHUNK
CB = 8192
NB = K // CB
KEY_MIN = -2147483648
KEY_MAX = 2147483647
OUT_PAD = 128


def _to_key(x):
    bits = jax.lax.bitcast_convert_type(x, jnp.int32)
    return jnp.where(bits < 0, bits ^ jnp.int32(0x7FFFFFFF), bits)


def _from_key(k):
    bits = jnp.where(k < 0, k ^ jnp.int32(0x7FFFFFFF), k)
    return jax.lax.bitcast_convert_type(bits, jnp.float32)


def _body(feat_ref, labels_ref, lblq_ref, fq_ref, cdw_ref, cdb_ref, cow_ref,
          cob_ref, kdw_ref, kdb_ref, kow_ref, kob_ref, loss_ref, upos_ref,
          uneg_ref, lq_ref, lcls_ref, cacc_ref, mp_ref, mn_ref):
    i = pl.program_id(0)

    @pl.when(i == 0)
    def _heads():
        f = feat_ref[...]
        h = jnp.tanh(
            jax.lax.dot_general(f, kdw_ref[...], (((1,), (0,)), ((), ())),
                                preferred_element_type=jnp.float32)
            + kdb_ref[...])
        logits = jax.lax.dot_general(h, kow_ref[...], (((1,), (0,)), ((), ())),
                                     preferred_element_type=jnp.float32)
        logits = logits + kob_ref[...]
        col = jax.lax.broadcasted_iota(jnp.int32, (B, OUT_PAD), 1)
        logits = jnp.where(col < NUM_LABELS, logits, -3e38)
        m = jnp.max(logits, axis=1, keepdims=True)
        lse = jnp.log(jnp.sum(jnp.exp(logits - m), axis=1, keepdims=True)) + m
        pick = jnp.sum(jnp.where(col == labels_ref[...], logits, 0.0),
                       axis=1, keepdims=True)
        lcls_ref[...] = jnp.reshape(jnp.mean(lse - pick), (1, 1))
        c = jnp.tanh(
            jax.lax.dot_general(f, cdw_ref[...], (((1,), (0,)), ((), ())),
                                preferred_element_type=jnp.float32)
            + cdb_ref[...])
        q = jax.lax.dot_general(c, cow_ref[...], (((1,), (0,)), ((), ())),
                                preferred_element_type=jnp.float32)
        q = q + cob_ref[...]
        lq_ref[...] = q / jnp.sqrt(jnp.sum(q * q, axis=1, keepdims=True))
        cacc_ref[...] = jnp.zeros((B, 1), jnp.float32)
        mp_ref[...] = jnp.full((B, 1), KEY_MIN, jnp.int32)
        mn_ref[...] = jnp.full((B, 1), KEY_MIN, jnp.int32)

    cos = jax.lax.dot_general(lq_ref[...], fq_ref[...],
                              (((1,), (1,)), ((), ())),
                              preferred_element_type=jnp.float32)
    key = _to_key(cos)
    off = i * CHUNK
    mask = lblq_ref[:, pl.ds(off, CHUNK)] == labels_ref[...]
    kp = jnp.where(mask, key, KEY_MIN)
    kn = jnp.where(mask, KEY_MIN, key)
    upos_ref[:, pl.ds(off, CHUNK)] = kp
    uneg_ref[:, pl.ds(off, CHUNK)] = kn
    cacc_ref[...] += jnp.sum(jnp.where(mask, 1.0, 0.0), axis=1, keepdims=True)
    mp_ref[...] = jnp.maximum(mp_ref[...], jnp.max(kp, axis=1, keepdims=True))
    mn_ref[...] = jnp.maximum(mn_ref[...], jnp.max(kn, axis=1, keepdims=True))

    @pl.when(i == NCHUNK - 1)
    def _selection():
        z = jnp.zeros((B, 1), jnp.float32)

        cpos = cacc_ref[...]
        pos_min = jnp.min(cpos)
        neg_min = jnp.min(float(K) - cpos)
        start = jnp.clip(pos_min - END_K, 0.0, float(K - END_K))
        kpos = start + END_K
        kneg = neg_min

        pp = jnp.full((B, 1), KEY_MIN, jnp.int32)
        pn = jnp.full((B, 1), KEY_MIN, jnp.int32)
        for bit in range(30, -1, -1):
            step = jnp.int32(1 << bit)
            tp = pp + step
            tn = pn + step
            def blk(j, c, tp=tp, tn=tn):
                cp, cn = c
                up = upos_ref[:, pl.ds(j * CB, CB)]
                un = uneg_ref[:, pl.ds(j * CB, CB)]
                cp = cp + jnp.sum(jnp.where(up >= tp, 1.0, 0.0), axis=1,
                                  keepdims=True)
                cn = cn + jnp.sum(jnp.where(un >= tn, 1.0, 0.0), axis=1,
                                  keepdims=True)
                return cp, cn
            cp, cn = jax.lax.fori_loop(0, NB, blk, (z, z))
            pp = jnp.where(cp >= kpos, tp, pp)
            pn = jnp.where(cn >= kneg, tn, pn)
        tpos, tneg = pp, pn

        def passa(j, c):
            gp, gn, n0 = c
            up = upos_ref[:, pl.ds(j * CB, CB)]
            un = uneg_ref[:, pl.ds(j * CB, CB)]
            gp = gp + jnp.sum(jnp.where(up > tpos, 1.0, 0.0), axis=1,
                              keepdims=True)
            gn = gn + jnp.sum(jnp.where(un > tneg, 1.0, 0.0), axis=1,
                              keepdims=True)
            n0 = jnp.minimum(n0, jnp.min(
                jnp.where(up > tpos, up, KEY_MAX), axis=1, keepdims=True))
            return gp, gn, n0
        gpos, gneg, n_cur = jax.lax.fori_loop(
            0, NB, passa, (z, z, jnp.full((B, 1), KEY_MAX, jnp.int32)))
        m_cur = mp_ref[...]
        M = _from_key(jnp.maximum(mn_ref[...], m_cur))

        def passb(j, e):
            un = uneg_ref[:, pl.ds(j * CB, CB)]
            x = _from_key(un)
            t = jnp.where(un > tneg, jnp.exp((x - M) / T), 0.0)
            return e + jnp.sum(t, axis=1, keepdims=True)
        esum = jax.lax.fori_loop(0, NB, passb, z)
        tneg_val = _from_key(tneg)
        esum = esum + (kneg - gneg) * jnp.exp((tneg_val - M) / T)

        mvals, mcnts, nvals, ncnts = [], [], [], []
        for _ in range(TOP_K):
            def passc(j, c, m_cur=m_cur, n_cur=n_cur):
                cm, nm, cn_, nn = c
                up = upos_ref[:, pl.ds(j * CB, CB)]
                cm = cm + jnp.sum(jnp.where(up == m_cur, 1.0, 0.0), axis=1,
                                  keepdims=True)
                nm = jnp.maximum(nm, jnp.max(
                    jnp.where(up < m_cur, up, KEY_MIN), axis=1,
                    keepdims=True))
                cn_ = cn_ + jnp.sum(jnp.where(up == n_cur, 1.0, 0.0), axis=1,
                                    keepdims=True)
                nn = jnp.minimum(nn, jnp.min(
                    jnp.where(up > n_cur, up, KEY_MAX), axis=1,
                    keepdims=True))
                return cm, nm, cn_, nn
            cm, m_next, cn_, n_next = jax.lax.fori_loop(
                0, NB, passc,
                (z, jnp.full((B, 1), KEY_MIN, jnp.int32), z,
                 jnp.full((B, 1), KEY_MAX, jnp.int32)))
            mvals.append(m_cur); mcnts.append(cm)
            nvals.append(n_cur); ncnts.append(cn_)
            m_cur, n_cur = m_next, n_next

        mval = jnp.concatenate(mvals, axis=1)
        nval = jnp.concatenate(nvals, axis=1)
        mcum_l, ncum_l = [], []
        racc_m = jnp.zeros((B, 1), jnp.float32)
        racc_n = jnp.zeros((B, 1), jnp.float32)
        for q in range(TOP_K):
            racc_m = racc_m + mcnts[q]
            racc_n = racc_n + ncnts[q]
            mcum_l.append(racc_m)
            ncum_l.append(racc_n)
        mcum = jnp.concatenate(mcum_l, axis=1)
        ncum = jnp.concatenate(ncum_l, axis=1)
        mprev = mcum - jnp.concatenate(mcnts, axis=1)
        nprev = ncum - jnp.concatenate(ncnts, axis=1)
        mf = _from_key(mval)
        nf = _from_key(nval)
        tpos_val = _from_key(tpos)
        mt = kpos - gpos

        contrib = jnp.zeros((B, 1), jnp.float32)
        for j in range(TOP_K):
            jf = float(j)
            selm = (mprev <= jf) & (mcum > jf)
            vj = jnp.sum(jnp.where(selm, mf, 0.0), axis=1, keepdims=True)
            d = (vj - M) / T
            contrib = contrib + jnp.log(jnp.exp(d) + esum) - d
            seln = (nprev <= jf) & (ncum > jf)
            aj = jnp.sum(jnp.where(seln, nf, 0.0), axis=1, keepdims=True)
            wj = jnp.where(jf < END_K - mt, aj, tpos_val)
            d2 = (wj - M) / T
            contrib = contrib + jnp.log(jnp.exp(d2) + esum) - d2
        lcon = jnp.sum(contrib) / (B * (TOP_K + END_K))
        loss_ref[...] = jnp.reshape(
            RATE * lcon + (1.0 - RATE) * lcls_ref[0, 0], (1, 1))


@jax.jit
def kernel(features, feature_queue, labels, label_queue, cls_dense_w,
           cls_dense_b, cls_out_w, cls_out_b, con_dense_w, con_dense_b,
           con_out_w, con_out_b):
    labels2 = labels.astype(jnp.int32).reshape(B, 1)
    lblq = label_queue.astype(jnp.int32).reshape(1, K)
    kow = jnp.zeros((HIDDEN, OUT_PAD), jnp.float32).at[:, :NUM_LABELS].set(
        cls_out_w)
    kob = jnp.zeros((1, OUT_PAD), jnp.float32).at[0, :NUM_LABELS].set(
        cls_out_b)

    const = lambda i: (0, 0)
    loss = pl.pallas_call(
        _body,
        grid=(NCHUNK,),
        in_specs=[
            pl.BlockSpec((B, HIDDEN), const),
            pl.BlockSpec((B, 1), const),
            pl.BlockSpec((1, K), const),
            pl.BlockSpec((CHUNK, HIDDEN), lambda i: (i, 0)),
            pl.BlockSpec((HIDDEN, HIDDEN), const),
            pl.BlockSpec((1, HIDDEN), const),
            pl.BlockSpec((HIDDEN, HIDDEN), const),
            pl.BlockSpec((1, HIDDEN), const),
            pl.BlockSpec((HIDDEN, HIDDEN), const),
            pl.BlockSpec((1, HIDDEN), const),
            pl.BlockSpec((HIDDEN, OUT_PAD), const),
            pl.BlockSpec((1, OUT_PAD), const),
        ],
        out_specs=pl.BlockSpec((1, 1), const),
        out_shape=jax.ShapeDtypeStruct((1, 1), jnp.float32),
        scratch_shapes=[pltpu.VMEM((B, K), jnp.int32),
                        pltpu.VMEM((B, K), jnp.int32),
                        pltpu.VMEM((B, HIDDEN), jnp.float32),
                        pltpu.VMEM((1, 1), jnp.float32),
                        pltpu.VMEM((B, 1), jnp.float32),
                        pltpu.VMEM((B, 1), jnp.int32),
                        pltpu.VMEM((B, 1), jnp.int32)],
        compiler_params=pltpu.CompilerParams(
            dimension_semantics=("arbitrary",)),
    )(features, labels2, lblq, feature_queue, con_dense_w,
      con_dense_b.reshape(1, HIDDEN), con_out_w, con_out_b.reshape(1, HIDDEN),
      cls_dense_w, cls_dense_b.reshape(1, HIDDEN), kow, kob)

    return loss[0, 0].reshape(())


# speedup vs baseline: 1.2245x; 1.0048x over previous
import jax
import jax.numpy as jnp
from jax.experimental import pallas as pl
from jax.experimental.pallas import tpu as pltpu

B = 64
K = 32768
HIDDEN = 1024
NUM_LABELS = 3
TOP_K = 8
END_K = 8
T = 0.07
RATE = 0.5

CHUNK = 2048
NCHUNK = K // CHUNK
CB = 8192
NB = K // CB
KEY_MIN = -2147483648
KEY_MAX = 2147483647
OUT_PAD = 128


def _to_key(x):
    bits = jax.lax.bitcast_convert_type(x, jnp.int32)
    return jnp.where(bits < 0, bits ^ jnp.int32(0x7FFFFFFF), bits)


def _from_key(k):
    bits = jnp.where(k < 0, k ^ jnp.int32(0x7FFFFFFF), k)
    return jax.lax.bitcast_convert_type(bits, jnp.float32)


def _body(feat_ref, labels_ref, lblq_ref, fq_ref, cdw_ref, cdb_ref, cow_ref,
          cob_ref, kdw_ref, kdb_ref, kow_ref, kob_ref, loss_ref, upos_ref,
          uneg_ref, lq_ref, lcls_ref, cacc_ref, mp_ref, mn_ref):
    i = pl.program_id(0)

    @pl.when(i == 0)
    def _heads():
        f = feat_ref[...]
        h = jnp.tanh(
            jax.lax.dot_general(f, kdw_ref[...], (((1,), (0,)), ((), ())),
                                preferred_element_type=jnp.float32)
            + kdb_ref[...])
        logits = jax.lax.dot_general(h, kow_ref[...], (((1,), (0,)), ((), ())),
                                     preferred_element_type=jnp.float32)
        logits = logits + kob_ref[...]
        col = jax.lax.broadcasted_iota(jnp.int32, (B, OUT_PAD), 1)
        logits = jnp.where(col < NUM_LABELS, logits, -3e38)
        m = jnp.max(logits, axis=1, keepdims=True)
        lse = jnp.log(jnp.sum(jnp.exp(logits - m), axis=1, keepdims=True)) + m
        pick = jnp.sum(jnp.where(col == labels_ref[...], logits, 0.0),
                       axis=1, keepdims=True)
        lcls_ref[...] = jnp.reshape(jnp.mean(lse - pick), (1, 1))
        c = jnp.tanh(
            jax.lax.dot_general(f, cdw_ref[...], (((1,), (0,)), ((), ())),
                                preferred_element_type=jnp.float32)
            + cdb_ref[...])
        q = jax.lax.dot_general(c, cow_ref[...], (((1,), (0,)), ((), ())),
                                preferred_element_type=jnp.float32)
        q = q + cob_ref[...]
        lq_ref[...] = q / jnp.sqrt(jnp.sum(q * q, axis=1, keepdims=True))
        cacc_ref[...] = jnp.zeros((B, 1), jnp.float32)
        mp_ref[...] = jnp.full((B, 1), KEY_MIN, jnp.int32)
        mn_ref[...] = jnp.full((B, 1), KEY_MIN, jnp.int32)

    cos = jax.lax.dot_general(lq_ref[...], fq_ref[...],
                              (((1,), (1,)), ((), ())),
                              preferred_element_type=jnp.float32)
    key = _to_key(cos)
    off = i * CHUNK
    mask = lblq_ref[:, pl.ds(off, CHUNK)] == labels_ref[...]
    kp = jnp.where(mask, key, KEY_MIN)
    kn = jnp.where(mask, KEY_MIN, key)
    upos_ref[:, pl.ds(off, CHUNK)] = kp
    uneg_ref[:, pl.ds(off, CHUNK)] = kn
    cacc_ref[...] += jnp.sum(jnp.where(mask, 1.0, 0.0), axis=1, keepdims=True)
    mp_ref[...] = jnp.maximum(mp_ref[...], jnp.max(kp, axis=1, keepdims=True))
    mn_ref[...] = jnp.maximum(mn_ref[...], jnp.max(kn, axis=1, keepdims=True))

    @pl.when(i == NCHUNK - 1)
    def _selection():
        z = jnp.zeros((B, 1), jnp.float32)

        cpos = cacc_ref[...]
        pos_min = jnp.min(cpos)
        neg_min = jnp.min(float(K) - cpos)
        start = jnp.clip(pos_min - END_K, 0.0, float(K - END_K))
        kpos = start + END_K
        kneg = neg_min

        pp = jnp.full((B, 1), KEY_MIN, jnp.int32)
        pn = jnp.full((B, 1), KEY_MIN, jnp.int32)
        for bit in range(30, -1, -1):
            step = jnp.int32(1 << bit)
            tp = pp + step
            tn = pn + step
            def blk(j, c, tp=tp, tn=tn):
                cp, cn = c
                up = upos_ref[:, pl.ds(j * CB, CB)]
                un = uneg_ref[:, pl.ds(j * CB, CB)]
                cp = cp + jnp.sum(jnp.where(up >= tp, 1.0, 0.0), axis=1,
                                  keepdims=True)
                cn = cn + jnp.sum(jnp.where(un >= tn, 1.0, 0.0), axis=1,
                                  keepdims=True)
                return cp, cn
            cp, cn = jax.lax.fori_loop(0, NB, blk, (z, z))
            pp = jnp.where(cp >= kpos, tp, pp)
            pn = jnp.where(cn >= kneg, tn, pn)
        tpos, tneg = pp, pn

        m_cur = mp_ref[...]
        M = _from_key(jnp.maximum(mn_ref[...], m_cur))

        def passa(j, c):
            gp, gn, n0, e = c
            up = upos_ref[:, pl.ds(j * CB, CB)]
            un = uneg_ref[:, pl.ds(j * CB, CB)]
            gp = gp + jnp.sum(jnp.where(up > tpos, 1.0, 0.0), axis=1,
                              keepdims=True)
            sel = un > tneg
            gn = gn + jnp.sum(jnp.where(sel, 1.0, 0.0), axis=1, keepdims=True)
            x = _from_key(un)
            e = e + jnp.sum(jnp.where(sel, jnp.exp((x - M) / T), 0.0),
                            axis=1, keepdims=True)
            n0 = jnp.minimum(n0, jnp.min(
                jnp.where(up > tpos, up, KEY_MAX), axis=1, keepdims=True))
            return gp, gn, n0, e
        gpos, gneg, n_cur, esum = jax.lax.fori_loop(
            0, NB, passa, (z, z, jnp.full((B, 1), KEY_MAX, jnp.int32), z))
        tneg_val = _from_key(tneg)
        esum = esum + (kneg - gneg) * jnp.exp((tneg_val - M) / T)

        mvals, mcnts, nvals, ncnts = [], [], [], []
        for _ in range(TOP_K):
            def passc(j, c, m_cur=m_cur, n_cur=n_cur):
                cm, nm, cn_, nn = c
                up = upos_ref[:, pl.ds(j * CB, CB)]
                cm = cm + jnp.sum(jnp.where(up == m_cur, 1.0, 0.0), axis=1,
                                  keepdims=True)
                nm = jnp.maximum(nm, jnp.max(
                    jnp.where(up < m_cur, up, KEY_MIN), axis=1,
                    keepdims=True))
                cn_ = cn_ + jnp.sum(jnp.where(up == n_cur, 1.0, 0.0), axis=1,
                                    keepdims=True)
                nn = jnp.minimum(nn, jnp.min(
                    jnp.where(up > n_cur, up, KEY_MAX), axis=1,
                    keepdims=True))
                return cm, nm, cn_, nn
            cm, m_next, cn_, n_next = jax.lax.fori_loop(
                0, NB, passc,
                (z, jnp.full((B, 1), KEY_MIN, jnp.int32), z,
                 jnp.full((B, 1), KEY_MAX, jnp.int32)))
            mvals.append(m_cur); mcnts.append(cm)
            nvals.append(n_cur); ncnts.append(cn_)
            m_cur, n_cur = m_next, n_next

        mval = jnp.concatenate(mvals, axis=1)
        nval = jnp.concatenate(nvals, axis=1)
        mcum_l, ncum_l = [], []
        racc_m = jnp.zeros((B, 1), jnp.float32)
        racc_n = jnp.zeros((B, 1), jnp.float32)
        for q in range(TOP_K):
            racc_m = racc_m + mcnts[q]
            racc_n = racc_n + ncnts[q]
            mcum_l.append(racc_m)
            ncum_l.append(racc_n)
        mcum = jnp.concatenate(mcum_l, axis=1)
        ncum = jnp.concatenate(ncum_l, axis=1)
        mprev = mcum - jnp.concatenate(mcnts, axis=1)
        nprev = ncum - jnp.concatenate(ncnts, axis=1)
        mf = _from_key(mval)
        nf = _from_key(nval)
        tpos_val = _from_key(tpos)
        mt = kpos - gpos

        contrib = jnp.zeros((B, 1), jnp.float32)
        for j in range(TOP_K):
            jf = float(j)
            selm = (mprev <= jf) & (mcum > jf)
            vj = jnp.sum(jnp.where(selm, mf, 0.0), axis=1, keepdims=True)
            d = (vj - M) / T
            contrib = contrib + jnp.log(jnp.exp(d) + esum) - d
            seln = (nprev <= jf) & (ncum > jf)
            aj = jnp.sum(jnp.where(seln, nf, 0.0), axis=1, keepdims=True)
            wj = jnp.where(jf < END_K - mt, aj, tpos_val)
            d2 = (wj - M) / T
            contrib = contrib + jnp.log(jnp.exp(d2) + esum) - d2
        lcon = jnp.sum(contrib) / (B * (TOP_K + END_K))
        loss_ref[...] = jnp.reshape(
            RATE * lcon + (1.0 - RATE) * lcls_ref[0, 0], (1, 1))


@jax.jit
def kernel(features, feature_queue, labels, label_queue, cls_dense_w,
           cls_dense_b, cls_out_w, cls_out_b, con_dense_w, con_dense_b,
           con_out_w, con_out_b):
    labels2 = labels.astype(jnp.int32).reshape(B, 1)
    lblq = label_queue.astype(jnp.int32).reshape(1, K)
    kow = jnp.zeros((HIDDEN, OUT_PAD), jnp.float32).at[:, :NUM_LABELS].set(
        cls_out_w)
    kob = jnp.zeros((1, OUT_PAD), jnp.float32).at[0, :NUM_LABELS].set(
        cls_out_b)

    const = lambda i: (0, 0)
    loss = pl.pallas_call(
        _body,
        grid=(NCHUNK,),
        in_specs=[
            pl.BlockSpec((B, HIDDEN), const),
            pl.BlockSpec((B, 1), const),
            pl.BlockSpec((1, K), const),
            pl.BlockSpec((CHUNK, HIDDEN), lambda i: (i, 0)),
            pl.BlockSpec((HIDDEN, HIDDEN), const),
            pl.BlockSpec((1, HIDDEN), const),
            pl.BlockSpec((HIDDEN, HIDDEN), const),
            pl.BlockSpec((1, HIDDEN), const),
            pl.BlockSpec((HIDDEN, HIDDEN), const),
            pl.BlockSpec((1, HIDDEN), const),
            pl.BlockSpec((HIDDEN, OUT_PAD), const),
            pl.BlockSpec((1, OUT_PAD), const),
        ],
        out_specs=pl.BlockSpec((1, 1), const),
        out_shape=jax.ShapeDtypeStruct((1, 1), jnp.float32),
        scratch_shapes=[pltpu.VMEM((B, K), jnp.int32),
                        pltpu.VMEM((B, K), jnp.int32),
                        pltpu.VMEM((B, HIDDEN), jnp.float32),
                        pltpu.VMEM((1, 1), jnp.float32),
                        pltpu.VMEM((B, 1), jnp.float32),
                        pltpu.VMEM((B, 1), jnp.int32),
                        pltpu.VMEM((B, 1), jnp.int32)],
        compiler_params=pltpu.CompilerParams(
            dimension_semantics=("arbitrary",)),
    )(features, labels2, lblq, feature_queue, con_dense_w,
      con_dense_b.reshape(1, HIDDEN), con_out_w, con_out_b.reshape(1, HIDDEN),
      cls_dense_w, cls_dense_b.reshape(1, HIDDEN), kow, kob)

    return loss[0, 0].reshape(())
